# Initial kernel scaffold; baseline (speedup 1.0000x reference)
import jax
import jax.numpy as jnp
from jax.experimental import pallas as pl


def kernel(x, x_0, t_code, bi_v_idx, bi_edge_src, bi_edge_dst, bi_edge_length, bi_uv_rotation, coarse_edge_src, coarse_edge_dst, coarse_edge_length, coarse_e_rotation, d_Wq, d_Wk, d_Wv, d_We, d_Wt, d_Wo, d_Wo0, g_Wq, g_Wk, g_Wv, g_We, g_Wt, g_Wo, g_Wo0):
    raise NotImplementedError("write your pallas kernel here")


# speedup vs baseline: 1.0187x; 1.0187x over previous
import functools
import jax
import jax.numpy as jnp
from jax.experimental import pallas as pl
from jax.experimental.pallas import tpu as pltpu

N_FINE = 10000
N_COARSE = 2500
E_BI = 160000
E_C = 40000
C = 256
H = 8
DH = C // H
EDGE_DIM = 64
MAX_LEN = 1.0
MAX_LEN_BI = 1.0


def _mm_kernel(a_ref, b_ref, o_ref):
    o_ref[...] = jnp.dot(a_ref[...], b_ref[...],
                         preferred_element_type=jnp.float32)


def _mm(a, b, block_rows):
    m, k = a.shape
    n = b.shape[1]
    grid = (m // block_rows,)
    return pl.pallas_call(
        _mm_kernel,
        grid=grid,
        in_specs=[
            pl.BlockSpec((block_rows, k), lambda i: (i, 0)),
            pl.BlockSpec((k, n), lambda i: (0, 0)),
        ],
        out_specs=pl.BlockSpec((block_rows, n), lambda i: (i, 0)),
        out_shape=jax.ShapeDtypeStruct((m, n), jnp.float32),
    )(a, b)


def _sinusoidal_emb(t, dim, max_val):
    t = t / max_val
    half = dim // 2
    emb = jnp.log(10000.0) / (half - 1)
    emb = jnp.exp(jnp.arange(half, dtype=jnp.float32) * -emb)
    emb = t[:, None] * emb[None, :]
    return jnp.concatenate([jnp.sin(emb), jnp.cos(emb)], axis=-1)


def _block(x_src, x_tgt, x0_src, x0_tgt, src, dst, n_dst, edge_scalar,
           t_code, R, Wq, Wk, Wv, We, Wt, Wo, Wo0, src_rows):
    q = (_mm(x_tgt, Wq, 2500))[dst].reshape(-1, H, DH)
    k = (_mm(x_src, Wk, src_rows))[src].reshape(-1, H, DH)
    v = (_mm(x_src, Wv, src_rows))[src].reshape(-1, H, DH)
    logits = jnp.sum(q * k, axis=-1) / jnp.sqrt(float(DH)) + edge_scalar @ We
    ex = jnp.exp(logits)
    den = jax.ops.segment_sum(ex, dst, num_segments=n_dst)
    num = jax.ops.segment_sum(ex[:, :, None] * v, dst, num_segments=n_dst)
    msg = (num / (den[:, :, None] + 1e-30)).reshape(n_dst, C)
    x0_rot = jnp.einsum('eij,ejc->eic', R, x0_src[src])
    w = jnp.mean(ex / (den[dst] + 1e-30), axis=-1)
    msg0 = jax.ops.segment_sum(w[:, None, None] * x0_rot, dst,
                               num_segments=n_dst)
    gate = jax.nn.silu(t_code @ Wt)
    y = x_tgt + _mm(msg * gate[None, :], Wo, 2500)
    y0 = x0_tgt + jnp.stack(
        [_mm(msg0[:, i, :], Wo0, 2500) for i in range(3)], axis=1)
    return y, y0


def kernel(x, x_0, t_code, bi_v_idx, bi_edge_src, bi_edge_dst,
           bi_edge_length, bi_uv_rotation, coarse_edge_src, coarse_edge_dst,
           coarse_edge_length, coarse_e_rotation, d_Wq, d_Wk, d_Wv, d_We,
           d_Wt, d_Wo, d_Wo0, g_Wq, g_Wk, g_Wv, g_We, g_Wt, g_Wo, g_Wo0):
    pre_v_f = x[bi_v_idx]
    pre_v_f_0 = x_0[bi_v_idx]
    bi_es = _sinusoidal_emb(bi_edge_length, EDGE_DIM, MAX_LEN_BI)
    c_es = _sinusoidal_emb(coarse_edge_length, EDGE_DIM, MAX_LEN)
    x_down, x_down_0 = _block(
        x, pre_v_f, x_0, pre_v_f_0, bi_edge_src, bi_edge_dst, N_COARSE,
        bi_es, t_code, bi_uv_rotation, d_Wq, d_Wk, d_Wv, d_We, d_Wt, d_Wo,
        d_Wo0, 2000)
    y, y_0 = _block(
        x_down, x_down, x_down_0, x_down_0, coarse_edge_src, coarse_edge_dst,
        N_COARSE, c_es, t_code, coarse_e_rotation, g_Wq, g_Wk, g_Wv, g_We,
        g_Wt, g_Wo, g_Wo0, 2500)
    return (y, y_0)
